# baseline (speedup 1.0000x reference)
import functools

import numpy as np
import jax
import jax.numpy as jnp
from jax.experimental import pallas as pl
from jax.experimental.pallas import tpu as pltpu
from jax.experimental.pallas import tpu_sc as plsc
from jax import lax

B = 1024
L = 20
V = 1000
ROW = L * V
NW = 32
NS = 16
TOTAL = B * ROW
PER_W = TOTAL // NW
CHUNK = 16000
NCHUNK = PER_W // CHUNK
NONES = B * L
ONES_PER_CORE = NONES // 2
ONES_PER_W = ONES_PER_CORE // NS

_Q = np.arange(NONES, dtype=np.int32)
_B = _Q // L
_COL0 = (_Q % L) * V
_BOFF = (_B // 128) * 1024 + (_B % 128)


def _ohe_body(zeros_hbm, offs_hbm, ones_hbm, out_hbm,
              offs_v, buf_v, ones_v, sem, sem2):
    cid = lax.axis_index("c")
    sid = lax.axis_index("s")
    wid = cid * NS + sid
    base = wid * PER_W

    pltpu.sync_copy(zeros_hbm, buf_v)
    copies = []
    for c in range(NCHUNK):
        copies.append(pltpu.async_copy(
            buf_v, out_hbm.at[pl.ds(base + c * CHUNK, CHUNK)], sem))

    ebase = cid * ONES_PER_CORE + sid * ONES_PER_W
    pltpu.sync_copy(offs_hbm.at[pl.ds(ebase, ONES_PER_W)], offs_v)
    pltpu.sync_copy(ones_hbm, ones_v)

    for cp in copies:
        cp.wait()
    plsc.subcore_barrier()

    pltpu.async_copy(ones_v, out_hbm.at[offs_v], sem2).wait()


def kernel(indices, eye):
    idx = indices.reshape(-1).astype(jnp.int32)
    col = jnp.asarray(_COL0) + idx
    poff = (col // 8) * 8192 + (col % 8) * 128 + jnp.asarray(_BOFF)
    p2 = poff.reshape(B, L)
    offs = jnp.concatenate([p2[:, :L // 2].T.reshape(-1),
                            p2[:, L // 2:].T.reshape(-1)])
    zeros = jnp.zeros((CHUNK,), jnp.float32)
    ones = jnp.ones((ONES_PER_W,), jnp.float32)

    mesh = plsc.VectorSubcoreMesh(core_axis_name="c", subcore_axis_name="s")
    run = functools.partial(
        pl.kernel,
        mesh=mesh,
        out_type=jax.ShapeDtypeStruct((TOTAL,), jnp.float32),
        scratch_types=[
            pltpu.VMEM((ONES_PER_W,), jnp.int32),
            pltpu.VMEM((CHUNK,), jnp.float32),
            pltpu.VMEM((ONES_PER_W,), jnp.float32),
            pltpu.SemaphoreType.DMA,
            pltpu.SemaphoreType.DMA,
        ],
    )(_ohe_body)
    out_flat = run(zeros, offs, ones)
    return (out_flat.reshape(ROW // 8, 8, 8, 128)
            .transpose(1, 3, 0, 2)
            .reshape(B, ROW))


# speedup vs baseline: 1.0192x; 1.0192x over previous
import functools

import numpy as np
import jax
import jax.numpy as jnp
from jax.experimental import pallas as pl
from jax.experimental.pallas import tpu as pltpu
from jax.experimental.pallas import tpu_sc as plsc
from jax import lax

B = 1024
L = 20
V = 1000
ROW = L * V
NW = 32
NS = 16
TOTAL = B * ROW
PER_W = TOTAL // NW
CHUNK = 10000
NCHUNK = PER_W // CHUNK
NONES = B * L
ONES_PER_CORE = NONES // 2
ONES_PER_W = ONES_PER_CORE // NS

_Q = np.arange(NONES, dtype=np.int32)
_B = _Q // L
_COL0 = (_Q % L) * V
_BOFF = (_B // 128) * 1024 + (_B % 128)


def _ohe_body(zeros_hbm, offs_hbm, ones_hbm, out_hbm,
              offs_v, buf_v, ones_v, sem, sem2):
    cid = lax.axis_index("c")
    sid = lax.axis_index("s")
    wid = cid * NS + sid
    base = wid * PER_W

    pltpu.sync_copy(zeros_hbm.at[pl.ds(wid * CHUNK, CHUNK)], buf_v)
    copies = []
    for c in range(NCHUNK):
        copies.append(pltpu.async_copy(
            buf_v, out_hbm.at[pl.ds(base + c * CHUNK, CHUNK)], sem))

    ebase = cid * ONES_PER_CORE + sid * ONES_PER_W
    pltpu.sync_copy(offs_hbm.at[pl.ds(ebase, ONES_PER_W)], offs_v)
    pltpu.sync_copy(ones_hbm, ones_v)

    for cp in copies:
        cp.wait()
    plsc.subcore_barrier()

    pltpu.async_copy(ones_v, out_hbm.at[offs_v], sem2).wait()


def kernel(indices, eye):
    idx = indices.reshape(-1).astype(jnp.int32)
    col = jnp.asarray(_COL0) + idx
    poff = (col // 8) * 8192 + (col % 8) * 128 + jnp.asarray(_BOFF)
    p2 = poff.reshape(B, L)
    offs = jnp.concatenate([p2[:, :L // 2].T.reshape(-1),
                            p2[:, L // 2:].T.reshape(-1)])
    zeros = jnp.zeros((NW * CHUNK,), jnp.float32)
    ones = jnp.ones((ONES_PER_W,), jnp.float32)

    mesh = plsc.VectorSubcoreMesh(core_axis_name="c", subcore_axis_name="s")
    run = functools.partial(
        pl.kernel,
        mesh=mesh,
        out_type=jax.ShapeDtypeStruct((TOTAL,), jnp.float32),
        scratch_types=[
            pltpu.VMEM((ONES_PER_W,), jnp.int32),
            pltpu.VMEM((CHUNK,), jnp.float32),
            pltpu.VMEM((ONES_PER_W,), jnp.float32),
            pltpu.SemaphoreType.DMA,
            pltpu.SemaphoreType.DMA,
        ],
    )(_ohe_body)
    out_flat = run(zeros, offs, ones)
    return (out_flat.reshape(ROW // 8, 8, 8, 128)
            .transpose(1, 3, 0, 2)
            .reshape(B, ROW))
